# baseline (speedup 1.0000x reference)
import functools

import jax
import jax.numpy as jnp
from jax import lax
from jax.experimental import pallas as pl
from jax.experimental.pallas import tpu as pltpu
from jax.experimental.pallas import tpu_sc as plsc

D_MSA = 128
VOCAB = 22
NUM_CORES = 2
NUM_SUBCORES = 16
NW = NUM_CORES * NUM_SUBCORES
CHUNK = 128
NBUF = 4


@functools.partial(jax.jit, static_argnames=("total",))
def _sc_gather(idx1d, table, *, total):
    per_w = total // NW
    steps = per_w // CHUNK
    assert steps % NBUF == 0 and steps >= 2 * NBUF
    mesh = plsc.VectorSubcoreMesh(core_axis_name="c", subcore_axis_name="s")

    @functools.partial(
        pl.kernel,
        mesh=mesh,
        out_type=jax.ShapeDtypeStruct((total, D_MSA), jnp.float32),
        scratch_types=[
            pltpu.VMEM((per_w,), jnp.int32),
            pltpu.VMEM_SHARED((VOCAB, D_MSA), jnp.float32),
            pltpu.VMEM((NBUF, CHUNK, D_MSA), jnp.float32),
            pltpu.SemaphoreType.DMA,
        ]
        + [pltpu.SemaphoreType.DMA] * NBUF,
    )
    def k(idx_hbm, table_hbm, out_hbm, idx_v, table_v, rows_v, gsem, *ssem):
        wid = lax.axis_index("s") * NUM_CORES + lax.axis_index("c")
        t_base = wid * per_w

        @pl.when(lax.axis_index("s") == 0)
        def _stage_table():
            pltpu.sync_copy(table_hbm, table_v)

        pltpu.sync_copy(idx_hbm.at[pl.ds(t_base, per_w)], idx_v)
        plsc.subcore_barrier()

        def issue_gather(step, buf):
            pltpu.async_copy(
                table_v.at[idx_v.at[pl.ds(step * CHUNK, CHUNK)]],
                rows_v.at[buf],
                gsem,
            )

        def wait_gather(buf):
            pltpu.make_async_copy(
                table_v.at[idx_v.at[pl.ds(0, CHUNK)]], rows_v.at[buf], gsem
            ).wait()

        def issue_scatter(step, buf):
            pltpu.async_copy(
                rows_v.at[buf],
                out_hbm.at[pl.ds(t_base + step * CHUNK, CHUNK)],
                ssem[buf],
            )

        def wait_scatter(buf):
            pltpu.make_async_copy(
                rows_v.at[buf], out_hbm.at[pl.ds(0, CHUNK)], ssem[buf]
            ).wait()

        def trip(t, first, last):
            for b in range(NBUF):
                s = NBUF * t + b
                wait_gather(b)
                issue_scatter(s, b)
                if not (first and b == 0):
                    wait_scatter((b - 1) % NBUF)
                if not last or b == 0:
                    issue_gather(s + NBUF - 1, (b - 1) % NBUF)
            return t

        for b in range(NBUF - 1):
            issue_gather(b, b)
        trip(0, True, False)
        lax.fori_loop(1, steps // NBUF - 1, lambda t, c: trip(t, False, False), 0)
        trip(steps // NBUF - 1, False, True)
        wait_scatter(NBUF - 1)

    return k(idx1d, table)


def kernel(msa_idx, embed):
    if msa_idx.ndim == 2:
        msa_idx = msa_idx[None]
    b, n, l = msa_idx.shape
    total = b * n * l
    idx1d = msa_idx.reshape(total)
    out = _sc_gather(idx1d, embed, total=total)
    return out.reshape(b, n, l, D_MSA)
